# baseline (speedup 1.0000x reference)
import dataclasses
import functools

import jax
import jax.numpy as jnp
from jax import lax
from jax.experimental import pallas as pl
from jax.experimental.pallas import tpu as pltpu
from jax.experimental.pallas import tpu_sc as plsc

E = 8
K = 2
TILE = 128
ALPHA = 0.01
N = 2048
D = 1024
F = 2048
NT = (K * N) // TILE + E
P = NT * TILE
NTP = 48

def _sc_compiler_params():
    cp = pltpu.CompilerParams()
    if "needs_layout_passes" in pltpu.CompilerParams.__dataclass_fields__:
        cp = dataclasses.replace(cp, needs_layout_passes=False)
    return cp


_NC = 2
_NS = 16
_L = 16


def _iota16():
    return lax.iota(jnp.int32, _L)


def _full16(v, dtype=jnp.int32):
    return jnp.full((_L,), v, dtype=dtype)


def _router_body(x_ref, wr_ref, logits_ref, eidx_ref, wts_ref, psum_ref):
    x = x_ref[...]
    wr = wr_ref[...]
    logits = lax.dot_general(
        x.astype(jnp.bfloat16), wr.astype(jnp.bfloat16),
        (((1,), (1,)), ((), ())),
        preferred_element_type=jnp.float32)
    logits_ref[...] = logits
    io = lax.broadcasted_iota(jnp.int32, logits.shape, 1)
    v0 = jnp.max(logits, axis=1, keepdims=True)
    i0 = jnp.min(jnp.where(logits == v0, io, E), axis=1, keepdims=True)
    m0 = io == i0
    l1 = jnp.where(m0, -jnp.inf, logits)
    v1 = jnp.max(l1, axis=1, keepdims=True)
    i1 = jnp.min(jnp.where(l1 == v1, io, E), axis=1, keepdims=True)
    ex = jnp.exp(v1 - v0)
    w0 = 1.0 / (1.0 + ex)
    w1 = ex / (1.0 + ex)
    eidx_ref[...] = jnp.concatenate([i0, i1], axis=1).T.astype(jnp.int32)
    wts_ref[...] = jnp.concatenate([w0, w1], axis=1).T
    pm = jnp.exp(logits - v0)
    probs = pm / jnp.sum(pm, axis=1, keepdims=True)
    psum = jnp.sum(probs, axis=0, keepdims=True)
    psum_ref[...] = jnp.concatenate(
        [psum, jnp.zeros((1, _L - E), jnp.float32)], axis=1)


def _router(xt, wr):
    return pl.pallas_call(
        _router_body,
        out_shape=(
            jax.ShapeDtypeStruct((N, E), jnp.float32),
            jax.ShapeDtypeStruct((K, N), jnp.int32),
            jax.ShapeDtypeStruct((K, N), jnp.float32),
            jax.ShapeDtypeStruct((1, _L), jnp.float32),
        ),
    )(xt, wr)


def _dispatch_body(eidx_hbm, xbf_hbm, psum_hbm, pos_hbm, te_hbm, xs_hbm,
                  aux_hbm, e2_vm, hist_vm, hists_all, shared, posbuf, tokbuf,
                  rowbuf, tebuf, auxbuf, psum_vm):
    c = lax.axis_index("c")
    s = lax.axis_index("s")
    mychunk = 2 * s + c
    iota = _iota16()

    pltpu.sync_copy(eidx_hbm.at[pl.ds(s * 256, 256)], e2_vm)
    for h in range(2):
        hv = jnp.zeros((_L,), jnp.int32)
        for v in range(8):
            ev = e2_vm[pl.ds(h * 128 + v * _L, _L)]
            for e in range(E):
                cnt = jnp.sum(jnp.where(ev == e, 1, 0))
                hv = hv + jnp.where(iota == e, _full16(cnt), _full16(0))
        hist_vm[h, pl.ds(0, _L)] = hv
    pltpu.sync_copy(hist_vm, shared.at[pl.ds(2 * s, 2)])
    plsc.subcore_barrier()

    pltpu.sync_copy(shared, hists_all)
    counts_vec = jnp.zeros((_L,), jnp.int32)
    prior_vec = jnp.zeros((_L,), jnp.int32)
    for w in range(32):
        hv = hists_all[w, pl.ds(0, _L)]
        counts_vec = counts_vec + hv
        take = jnp.full((_L,), w < mychunk)
        prior_vec = prior_vec + jnp.where(take, hv, _full16(0))
    cnt_s = [jnp.sum(jnp.where(iota == e, counts_vec, _full16(0)))
             for e in range(E)]
    prior_s = [jnp.sum(jnp.where(iota == e, prior_vec, _full16(0)))
               for e in range(E)]
    po = []
    run = jnp.int32(0)
    for e in range(E):
        po.append(run)
        aligned = ((cnt_s[e] + (TILE - 1)) >> 7) << 7
        run = run + aligned
    tbl = [po[e] + prior_s[e] for e in range(E)]

    for v in range(8):
        ev = e2_vm[pl.ds(c * 128 + v * _L, _L)]
        pos_v = jnp.zeros((_L,), jnp.int32)
        for e in range(E):
            m = ev == e
            mi = jnp.where(m, 1, 0).astype(jnp.int32)
            cs = jnp.cumsum(mi)
            pos_v = pos_v + jnp.where(m, _full16(tbl[e]) + cs - 1, _full16(0))
            tbl[e] = tbl[e] + jnp.sum(mi)
        posbuf[v // 2, pl.ds((v % 2) * _L, _L)] = pos_v
        tok_v = (mychunk * 128 + v * _L + iota) & (N - 1)
        tokbuf[v // 2, pl.ds((v % 2) * _L, _L)] = tok_v
    pltpu.sync_copy(posbuf, pos_hbm.at[mychunk])

    for g in range(4):
        pltpu.sync_copy(xbf_hbm.at[tokbuf.at[g]], rowbuf)
        pltpu.sync_copy(rowbuf, xs_hbm.at[posbuf.at[g]])

    @pl.when(jnp.logical_and(c == 0, s == 0))
    def _():
        for grp in range(NTP // _L):
            tv = (grp * _L + iota) * TILE
            acc = jnp.zeros((_L,), jnp.int32)
            for e in range(E):
                acc = acc + jnp.where(tv >= _full16(po[e]),
                                      _full16(1), _full16(0))
            tebuf[pl.ds(grp * _L, _L)] = acc - 1
        pltpu.sync_copy(tebuf, te_hbm)
        pltpu.sync_copy(psum_hbm, psum_vm)
        pv = psum_vm[0, pl.ds(0, _L)]
        cf = counts_vec.astype(jnp.float32)
        total = jnp.sum(cf * pv)
        auxbuf[pl.ds(0, _L)] = _full16(
            total * (ALPHA * E / (N * N)), jnp.float32)
        pltpu.sync_copy(auxbuf, aux_hbm)


def _dispatch(eidx_flat, xbf, psum):
    mesh = plsc.VectorSubcoreMesh(core_axis_name="c", subcore_axis_name="s")
    kern = pl.kernel(
        _dispatch_body,
        out_type=(
            jax.ShapeDtypeStruct((32, 4, 32), jnp.int32),
            jax.ShapeDtypeStruct((NTP,), jnp.int32),
            jax.ShapeDtypeStruct((P, D), jnp.float32),
            jax.ShapeDtypeStruct((_L,), jnp.float32),
        ),
        mesh=mesh,
        scratch_types=[
            pltpu.VMEM((256,), jnp.int32),
            pltpu.VMEM((2, _L), jnp.int32),
            pltpu.VMEM((32, _L), jnp.int32),
            pltpu.VMEM_SHARED((32, _L), jnp.int32),
            pltpu.VMEM((4, 32), jnp.int32),
            pltpu.VMEM((4, 32), jnp.int32),
            pltpu.VMEM((32, D), jnp.float32),
            pltpu.VMEM((NTP,), jnp.int32),
            pltpu.VMEM((_L,), jnp.float32),
            pltpu.VMEM((1, _L), jnp.float32),
        ],
        compiler_params=_sc_compiler_params(),
    )
    return kern(eidx_flat, xbf, psum)


def _ffn_body(te_ref, xs_ref, w1_ref, w3_ref, w2_ref, ys_ref):
    xb = xs_ref[...].astype(jnp.bfloat16)
    a = lax.dot_general(xb, w1_ref[0], (((1,), (1,)), ((), ())),
                        preferred_element_type=jnp.float32)
    b = lax.dot_general(xb, w3_ref[0], (((1,), (1,)), ((), ())),
                        preferred_element_type=jnp.float32)
    h = (a * jax.nn.sigmoid(a)) * b
    hb = h.astype(jnp.bfloat16)
    y = lax.dot_general(hb, w2_ref[0], (((1,), (1,)), ((), ())),
                        preferred_element_type=jnp.float32)
    ys_ref[...] = y


def _ffn(te, xs, w1b, w3b, w2b):
    grid_spec = pltpu.PrefetchScalarGridSpec(
        num_scalar_prefetch=1,
        grid=(NT,),
        in_specs=[
            pl.BlockSpec((TILE, D), lambda i, te: (i, 0)),
            pl.BlockSpec((1, F, D), lambda i, te: (te[i], 0, 0)),
            pl.BlockSpec((1, F, D), lambda i, te: (te[i], 0, 0)),
            pl.BlockSpec((1, D, F), lambda i, te: (te[i], 0, 0)),
        ],
        out_specs=pl.BlockSpec((TILE, D), lambda i, te: (i, 0)),
    )
    return pl.pallas_call(
        _ffn_body,
        grid_spec=grid_spec,
        out_shape=jax.ShapeDtypeStruct((P, D), jnp.float32),
        compiler_params=pltpu.CompilerParams(
            dimension_semantics=("arbitrary",)),
    )(te, xs, w1b, w3b, w2b)


def _combine_body(ys_hbm, pos_hbm, wts_hbm, out_hbm,
                  idx_vm, w_vm, abuf, bbuf):
    c = lax.axis_index("c")
    s = lax.axis_index("s")
    wid = 2 * s + c
    iota = _iota16()
    fz = jnp.zeros((_L,), jnp.float32)
    for g in range(2):
        tg = wid * 64 + g * 32
        pltpu.sync_copy(pos_hbm.at[pl.ds(tg, 32)], idx_vm.at[0])
        pltpu.sync_copy(pos_hbm.at[pl.ds(N + tg, 32)], idx_vm.at[1])
        pltpu.sync_copy(wts_hbm.at[pl.ds(tg, 32)], w_vm.at[0])
        pltpu.sync_copy(wts_hbm.at[pl.ds(N + tg, 32)], w_vm.at[1])
        pltpu.sync_copy(ys_hbm.at[idx_vm.at[0]], abuf)
        pltpu.sync_copy(ys_hbm.at[idx_vm.at[1]], bbuf)

        wv = [[w_vm[k, pl.ds(half * _L, _L)] for half in range(2)]
              for k in range(2)]
        for l in range(32):
            lane, half = l % _L, l // _L
            w0s = jnp.sum(jnp.where(iota == lane, wv[0][half], fz))
            w1s = jnp.sum(jnp.where(iota == lane, wv[1][half], fz))
            w0 = jnp.full((_L,), w0s, jnp.float32)
            w1 = jnp.full((_L,), w1s, jnp.float32)

            @pl.loop(0, D // _L)
            def _(cv, l=l, w0=w0, w1=w1):
                slc = pl.ds(cv * _L, _L)
                abuf[l, slc] = abuf[l, slc] * w0 + bbuf[l, slc] * w1

        pltpu.sync_copy(abuf, out_hbm.at[pl.ds(tg, 32)])


def _combine(ys, pos_flat, wts_flat):
    mesh = plsc.VectorSubcoreMesh(core_axis_name="c", subcore_axis_name="s")
    kern = pl.kernel(
        _combine_body,
        out_type=jax.ShapeDtypeStruct((N, D), jnp.float32),
        mesh=mesh,
        scratch_types=[
            pltpu.VMEM((2, 32), jnp.int32),
            pltpu.VMEM((2, 32), jnp.float32),
            pltpu.VMEM((32, D), jnp.float32),
            pltpu.VMEM((32, D), jnp.float32),
        ],
        compiler_params=_sc_compiler_params(),
    )
    return kern(ys, pos_flat, wts_flat)


def kernel(x, Wr, w1, w3, w2):
    B, T, Dm = x.shape
    xt = x.reshape(B * T, Dm)
    logits, eidx, wts, psum = _router(xt, Wr)
    pos, te, xs, aux = _dispatch(eidx.reshape(K * N), xt, psum)
    ys = _ffn(te, xs, w1.astype(jnp.bfloat16), w3.astype(jnp.bfloat16),
              w2.astype(jnp.bfloat16))
    out = _combine(ys, pos.reshape(K * N), wts.reshape(K * N))
    return (out.reshape(B, T, Dm), aux[0], logits.reshape(B, T, E))


# speedup vs baseline: 1.1721x; 1.1721x over previous
import dataclasses
import functools

import jax
import jax.numpy as jnp
from jax import lax
from jax.experimental import pallas as pl
from jax.experimental.pallas import tpu as pltpu
from jax.experimental.pallas import tpu_sc as plsc

E = 8
K = 2
TILE = 128
ALPHA = 0.01
N = 2048
D = 1024
F = 2048
NT = (K * N) // TILE + E
P = NT * TILE
NTP = 48

def _sc_compiler_params():
    cp = pltpu.CompilerParams()
    if "needs_layout_passes" in pltpu.CompilerParams.__dataclass_fields__:
        cp = dataclasses.replace(cp, needs_layout_passes=False)
    return cp


_NC = 2
_NS = 16
_L = 16


def _iota16():
    return lax.iota(jnp.int32, _L)


def _full16(v, dtype=jnp.int32):
    return jnp.full((_L,), v, dtype=dtype)


def _router_body(x_ref, wr_ref, logits_ref, eidx_ref, wts_ref, psum_ref):
    x = x_ref[...]
    wr = wr_ref[...]
    logits = lax.dot_general(
        x.astype(jnp.bfloat16), wr.astype(jnp.bfloat16),
        (((1,), (1,)), ((), ())),
        preferred_element_type=jnp.float32)
    logits_ref[...] = logits
    io = lax.broadcasted_iota(jnp.int32, logits.shape, 1)
    v0 = jnp.max(logits, axis=1, keepdims=True)
    i0 = jnp.min(jnp.where(logits == v0, io, E), axis=1, keepdims=True)
    m0 = io == i0
    l1 = jnp.where(m0, -jnp.inf, logits)
    v1 = jnp.max(l1, axis=1, keepdims=True)
    i1 = jnp.min(jnp.where(l1 == v1, io, E), axis=1, keepdims=True)
    ex = jnp.exp(v1 - v0)
    w0 = 1.0 / (1.0 + ex)
    w1 = ex / (1.0 + ex)
    eidx_ref[...] = jnp.concatenate([i0, i1], axis=1).T.astype(jnp.int32)
    wts_ref[...] = jnp.concatenate([w0, w1], axis=1).T
    pm = jnp.exp(logits - v0)
    probs = pm / jnp.sum(pm, axis=1, keepdims=True)
    psum = jnp.sum(probs, axis=0, keepdims=True)
    psum_ref[...] = jnp.concatenate(
        [psum, jnp.zeros((1, _L - E), jnp.float32)], axis=1)


def _router(xt, wr):
    return pl.pallas_call(
        _router_body,
        out_shape=(
            jax.ShapeDtypeStruct((N, E), jnp.float32),
            jax.ShapeDtypeStruct((K, N), jnp.int32),
            jax.ShapeDtypeStruct((K, N), jnp.float32),
            jax.ShapeDtypeStruct((1, _L), jnp.float32),
        ),
    )(xt, wr)


def _dispatch_body(eidx_hbm, xbf_hbm, psum_hbm, pos_hbm, te_hbm, xs_hbm,
                  aux_hbm, e2_vm, hist_vm, hists_all, shared, posbuf, tokbuf,
                  rowbuf, tebuf, auxbuf, psum_vm):
    c = lax.axis_index("c")
    s = lax.axis_index("s")
    mychunk = 2 * s + c
    iota = _iota16()

    pltpu.sync_copy(eidx_hbm.at[pl.ds(s * 256, 256)], e2_vm)
    for h in range(2):
        hv = jnp.zeros((_L,), jnp.int32)
        for v in range(8):
            ev = e2_vm[pl.ds(h * 128 + v * _L, _L)]
            for e in range(E):
                cnt = jnp.sum(jnp.where(ev == e, 1, 0))
                hv = hv + jnp.where(iota == e, _full16(cnt), _full16(0))
        hist_vm[h, pl.ds(0, _L)] = hv
    pltpu.sync_copy(hist_vm, shared.at[pl.ds(2 * s, 2)])
    plsc.subcore_barrier()

    pltpu.sync_copy(shared, hists_all)
    counts_vec = jnp.zeros((_L,), jnp.int32)
    prior_vec = jnp.zeros((_L,), jnp.int32)
    for w in range(32):
        hv = hists_all[w, pl.ds(0, _L)]
        counts_vec = counts_vec + hv
        take = jnp.full((_L,), w < mychunk)
        prior_vec = prior_vec + jnp.where(take, hv, _full16(0))
    cnt_s = [jnp.sum(jnp.where(iota == e, counts_vec, _full16(0)))
             for e in range(E)]
    prior_s = [jnp.sum(jnp.where(iota == e, prior_vec, _full16(0)))
               for e in range(E)]
    po = []
    run = jnp.int32(0)
    for e in range(E):
        po.append(run)
        aligned = ((cnt_s[e] + (TILE - 1)) >> 7) << 7
        run = run + aligned
    tbl = [po[e] + prior_s[e] for e in range(E)]

    for v in range(8):
        ev = e2_vm[pl.ds(c * 128 + v * _L, _L)]
        pos_v = jnp.zeros((_L,), jnp.int32)
        for e in range(E):
            m = ev == e
            mi = jnp.where(m, 1, 0).astype(jnp.int32)
            cs = jnp.cumsum(mi)
            pos_v = pos_v + jnp.where(m, _full16(tbl[e]) + cs - 1, _full16(0))
            tbl[e] = tbl[e] + jnp.sum(mi)
        posbuf[v // 2, pl.ds((v % 2) * _L, _L)] = pos_v
        tok_v = (mychunk * 128 + v * _L + iota) & (N - 1)
        tokbuf[v // 2, pl.ds((v % 2) * _L, _L)] = tok_v
    pltpu.sync_copy(posbuf, pos_hbm.at[mychunk])

    for g in range(4):
        pltpu.sync_copy(xbf_hbm.at[tokbuf.at[g]], rowbuf)
        pltpu.sync_copy(rowbuf, xs_hbm.at[posbuf.at[g]])

    @pl.when(jnp.logical_and(c == 0, s == 0))
    def _():
        for grp in range(NTP // _L):
            tv = (grp * _L + iota) * TILE
            acc = jnp.zeros((_L,), jnp.int32)
            for e in range(E):
                acc = acc + jnp.where(tv >= _full16(po[e]),
                                      _full16(1), _full16(0))
            tebuf[pl.ds(grp * _L, _L)] = acc - 1
        pltpu.sync_copy(tebuf, te_hbm)
        pltpu.sync_copy(psum_hbm, psum_vm)
        pv = psum_vm[0, pl.ds(0, _L)]
        cf = counts_vec.astype(jnp.float32)
        total = jnp.sum(cf * pv)
        auxbuf[pl.ds(0, _L)] = _full16(
            total * (ALPHA * E / (N * N)), jnp.float32)
        pltpu.sync_copy(auxbuf, aux_hbm)


def _dispatch(eidx_flat, xbf, psum):
    mesh = plsc.VectorSubcoreMesh(core_axis_name="c", subcore_axis_name="s")
    kern = pl.kernel(
        _dispatch_body,
        out_type=(
            jax.ShapeDtypeStruct((32, 4, 32), jnp.int32),
            jax.ShapeDtypeStruct((NTP,), jnp.int32),
            jax.ShapeDtypeStruct((P, D), jnp.float32),
            jax.ShapeDtypeStruct((_L,), jnp.float32),
        ),
        mesh=mesh,
        scratch_types=[
            pltpu.VMEM((256,), jnp.int32),
            pltpu.VMEM((2, _L), jnp.int32),
            pltpu.VMEM((32, _L), jnp.int32),
            pltpu.VMEM_SHARED((32, _L), jnp.int32),
            pltpu.VMEM((4, 32), jnp.int32),
            pltpu.VMEM((4, 32), jnp.int32),
            pltpu.VMEM((32, D), jnp.float32),
            pltpu.VMEM((NTP,), jnp.int32),
            pltpu.VMEM((_L,), jnp.float32),
            pltpu.VMEM((1, _L), jnp.float32),
        ],
        compiler_params=_sc_compiler_params(),
    )
    return kern(eidx_flat, xbf, psum)


def _ffn_body(te_ref, xs_ref, w1_ref, w3_ref, w2_ref, ys_ref):
    xb = xs_ref[...].astype(jnp.bfloat16)
    a = lax.dot_general(xb, w1_ref[0].astype(jnp.bfloat16),
                        (((1,), (1,)), ((), ())),
                        preferred_element_type=jnp.float32)
    b = lax.dot_general(xb, w3_ref[0].astype(jnp.bfloat16),
                        (((1,), (1,)), ((), ())),
                        preferred_element_type=jnp.float32)
    h = (a * jax.nn.sigmoid(a)) * b
    hb = h.astype(jnp.bfloat16)
    y = lax.dot_general(hb, w2_ref[0].astype(jnp.bfloat16),
                        (((1,), (1,)), ((), ())),
                        preferred_element_type=jnp.float32)
    ys_ref[...] = y


def _ffn(te, xs, w1b, w3b, w2b):
    grid_spec = pltpu.PrefetchScalarGridSpec(
        num_scalar_prefetch=1,
        grid=(NT,),
        in_specs=[
            pl.BlockSpec((TILE, D), lambda i, te: (i, 0)),
            pl.BlockSpec((1, F, D), lambda i, te: (te[i], 0, 0)),
            pl.BlockSpec((1, F, D), lambda i, te: (te[i], 0, 0)),
            pl.BlockSpec((1, D, F), lambda i, te: (te[i], 0, 0)),
        ],
        out_specs=pl.BlockSpec((TILE, D), lambda i, te: (i, 0)),
    )
    return pl.pallas_call(
        _ffn_body,
        grid_spec=grid_spec,
        out_shape=jax.ShapeDtypeStruct((P, D), jnp.float32),
        compiler_params=pltpu.CompilerParams(
            dimension_semantics=("arbitrary",)),
    )(te, xs, w1b, w3b, w2b)


def _combine_body(ys_hbm, pos_hbm, wts_hbm, out_hbm,
                  idx_vm, w_vm, abuf, bbuf):
    c = lax.axis_index("c")
    s = lax.axis_index("s")
    wid = 2 * s + c
    iota = _iota16()
    fz = jnp.zeros((_L,), jnp.float32)
    for g in range(2):
        tg = wid * 64 + g * 32
        pltpu.sync_copy(pos_hbm.at[pl.ds(tg, 32)], idx_vm.at[0])
        pltpu.sync_copy(pos_hbm.at[pl.ds(N + tg, 32)], idx_vm.at[1])
        pltpu.sync_copy(wts_hbm.at[pl.ds(tg, 32)], w_vm.at[0])
        pltpu.sync_copy(wts_hbm.at[pl.ds(N + tg, 32)], w_vm.at[1])
        pltpu.sync_copy(ys_hbm.at[idx_vm.at[0]], abuf)
        pltpu.sync_copy(ys_hbm.at[idx_vm.at[1]], bbuf)

        wv = [[w_vm[k, pl.ds(half * _L, _L)] for half in range(2)]
              for k in range(2)]
        for l in range(32):
            lane, half = l % _L, l // _L
            w0s = jnp.sum(jnp.where(iota == lane, wv[0][half], fz))
            w1s = jnp.sum(jnp.where(iota == lane, wv[1][half], fz))
            w0 = jnp.full((_L,), w0s, jnp.float32)
            w1 = jnp.full((_L,), w1s, jnp.float32)

            @pl.loop(0, D // _L)
            def _(cv, l=l, w0=w0, w1=w1):
                slc = pl.ds(cv * _L, _L)
                abuf[l, slc] = abuf[l, slc] * w0 + bbuf[l, slc] * w1

        pltpu.sync_copy(abuf, out_hbm.at[pl.ds(tg, 32)])


def _combine(ys, pos_flat, wts_flat):
    mesh = plsc.VectorSubcoreMesh(core_axis_name="c", subcore_axis_name="s")
    kern = pl.kernel(
        _combine_body,
        out_type=jax.ShapeDtypeStruct((N, D), jnp.float32),
        mesh=mesh,
        scratch_types=[
            pltpu.VMEM((2, 32), jnp.int32),
            pltpu.VMEM((2, 32), jnp.float32),
            pltpu.VMEM((32, D), jnp.float32),
            pltpu.VMEM((32, D), jnp.float32),
        ],
        compiler_params=_sc_compiler_params(),
    )
    return kern(ys, pos_flat, wts_flat)


def kernel(x, Wr, w1, w3, w2):
    B, T, Dm = x.shape
    xt = x.reshape(B * T, Dm)
    logits, eidx, wts, psum = _router(xt, Wr)
    pos, te, xs, aux = _dispatch(eidx.reshape(K * N), xt, psum)
    ys = _ffn(te, xs, w1, w3, w2)
    out = _combine(ys, pos.reshape(K * N), wts.reshape(K * N))
    return (out.reshape(B, T, Dm), aux[0], logits.reshape(B, T, E))


# speedup vs baseline: 1.2438x; 1.0611x over previous
import dataclasses
import functools

import jax
import jax.numpy as jnp
from jax import lax
from jax.experimental import pallas as pl
from jax.experimental.pallas import tpu as pltpu
from jax.experimental.pallas import tpu_sc as plsc

E = 8
K = 2
TILE = 128
ALPHA = 0.01
N = 2048
D = 1024
F = 2048
NT = (K * N) // TILE + E
P = NT * TILE
NTP = 48

def _sc_compiler_params():
    cp = pltpu.CompilerParams()
    if "needs_layout_passes" in pltpu.CompilerParams.__dataclass_fields__:
        cp = dataclasses.replace(cp, needs_layout_passes=False)
    return cp


_NC = 2
_NS = 16
_L = 16


def _iota16():
    return lax.iota(jnp.int32, _L)


def _full16(v, dtype=jnp.int32):
    return jnp.full((_L,), v, dtype=dtype)


def _router_body(x_ref, wr_ref, logits_ref, eidx_ref, wts_ref, psum_ref):
    x = x_ref[...]
    wr = wr_ref[...]
    logits = lax.dot_general(
        x.astype(jnp.bfloat16), wr.astype(jnp.bfloat16),
        (((1,), (1,)), ((), ())),
        preferred_element_type=jnp.float32)
    logits_ref[...] = logits
    io = lax.broadcasted_iota(jnp.int32, logits.shape, 1)
    v0 = jnp.max(logits, axis=1, keepdims=True)
    i0 = jnp.min(jnp.where(logits == v0, io, E), axis=1, keepdims=True)
    m0 = io == i0
    l1 = jnp.where(m0, -jnp.inf, logits)
    v1 = jnp.max(l1, axis=1, keepdims=True)
    i1 = jnp.min(jnp.where(l1 == v1, io, E), axis=1, keepdims=True)
    ex = jnp.exp(v1 - v0)
    w0 = 1.0 / (1.0 + ex)
    w1 = ex / (1.0 + ex)
    eidx_ref[...] = jnp.concatenate([i0, i1], axis=1).T.astype(jnp.int32)
    wts_ref[...] = jnp.concatenate([w0, w1], axis=1).T
    pm = jnp.exp(logits - v0)
    probs = pm / jnp.sum(pm, axis=1, keepdims=True)
    psum = jnp.sum(probs, axis=0, keepdims=True)
    psum_ref[...] = jnp.concatenate(
        [psum, jnp.zeros((1, _L - E), jnp.float32)], axis=1)


def _router(xt, wr):
    return pl.pallas_call(
        _router_body,
        out_shape=(
            jax.ShapeDtypeStruct((N, E), jnp.float32),
            jax.ShapeDtypeStruct((K, N), jnp.int32),
            jax.ShapeDtypeStruct((K, N), jnp.float32),
            jax.ShapeDtypeStruct((1, _L), jnp.float32),
        ),
    )(xt, wr)


def _dispatch_body(eidx_hbm, xbf_hbm, psum_hbm, pos_hbm, te_hbm, xs_hbm,
                  aux_hbm, e2_vm, hist_vm, hists_all, shared, posbuf, tokbuf,
                  rowbuf, tebuf, auxbuf, psum_vm, gsem, ssem):
    c = lax.axis_index("c")
    s = lax.axis_index("s")
    mychunk = 2 * s + c
    iota = _iota16()

    pltpu.sync_copy(eidx_hbm.at[pl.ds(s * 256, 256)], e2_vm)
    for h in range(2):
        hv = jnp.zeros((_L,), jnp.int32)
        for v in range(8):
            ev = e2_vm[pl.ds(h * 128 + v * _L, _L)]
            for e in range(E):
                cnt = jnp.sum(jnp.where(ev == e, 1, 0))
                hv = hv + jnp.where(iota == e, _full16(cnt), _full16(0))
        hist_vm[h, pl.ds(0, _L)] = hv
    pltpu.sync_copy(hist_vm, shared.at[pl.ds(2 * s, 2)])
    plsc.subcore_barrier()

    pltpu.sync_copy(shared, hists_all)
    counts_vec = jnp.zeros((_L,), jnp.int32)
    prior_vec = jnp.zeros((_L,), jnp.int32)
    for w in range(32):
        hv = hists_all[w, pl.ds(0, _L)]
        counts_vec = counts_vec + hv
        take = jnp.full((_L,), w < mychunk)
        prior_vec = prior_vec + jnp.where(take, hv, _full16(0))
    cnt_s = [jnp.sum(jnp.where(iota == e, counts_vec, _full16(0)))
             for e in range(E)]
    prior_s = [jnp.sum(jnp.where(iota == e, prior_vec, _full16(0)))
               for e in range(E)]
    po = []
    run = jnp.int32(0)
    for e in range(E):
        po.append(run)
        aligned = ((cnt_s[e] + (TILE - 1)) >> 7) << 7
        run = run + aligned
    tbl = [po[e] + prior_s[e] for e in range(E)]

    for v in range(8):
        ev = e2_vm[pl.ds(c * 128 + v * _L, _L)]
        pos_v = jnp.zeros((_L,), jnp.int32)
        for e in range(E):
            m = ev == e
            mi = jnp.where(m, 1, 0).astype(jnp.int32)
            cs = jnp.cumsum(mi)
            pos_v = pos_v + jnp.where(m, _full16(tbl[e]) + cs - 1, _full16(0))
            tbl[e] = tbl[e] + jnp.sum(mi)
        posbuf[v // 2, pl.ds((v % 2) * _L, _L)] = pos_v
        tok_v = (mychunk * 128 + v * _L + iota) & (N - 1)
        tokbuf[v // 2, pl.ds((v % 2) * _L, _L)] = tok_v
    pltpu.sync_copy(posbuf, pos_hbm.at[mychunk])

    pltpu.async_copy(xbf_hbm.at[tokbuf.at[0]], rowbuf.at[0], gsem).wait()
    for g in range(4):
        scp = pltpu.async_copy(rowbuf.at[g % 2], xs_hbm.at[posbuf.at[g]],
                               ssem)
        if g < 3:
            pltpu.async_copy(xbf_hbm.at[tokbuf.at[g + 1]],
                             rowbuf.at[(g + 1) % 2], gsem).wait()
        scp.wait()

    @pl.when(jnp.logical_and(c == 0, s == 0))
    def _():
        for grp in range(NTP // _L):
            tv = (grp * _L + iota) * TILE
            acc = jnp.zeros((_L,), jnp.int32)
            for e in range(E):
                acc = acc + jnp.where(tv >= _full16(po[e]),
                                      _full16(1), _full16(0))
            tebuf[pl.ds(grp * _L, _L)] = acc - 1
        pltpu.sync_copy(tebuf, te_hbm)
        pltpu.sync_copy(psum_hbm, psum_vm)
        pv = psum_vm[0, pl.ds(0, _L)]
        cf = counts_vec.astype(jnp.float32)
        total = jnp.sum(cf * pv)
        auxbuf[pl.ds(0, _L)] = _full16(
            total * (ALPHA * E / (N * N)), jnp.float32)
        pltpu.sync_copy(auxbuf, aux_hbm)


def _dispatch(eidx_flat, xbf, psum):
    mesh = plsc.VectorSubcoreMesh(core_axis_name="c", subcore_axis_name="s")
    kern = pl.kernel(
        _dispatch_body,
        out_type=(
            jax.ShapeDtypeStruct((32, 4, 32), jnp.int32),
            jax.ShapeDtypeStruct((NTP,), jnp.int32),
            jax.ShapeDtypeStruct((P, D), jnp.float32),
            jax.ShapeDtypeStruct((_L,), jnp.float32),
        ),
        mesh=mesh,
        scratch_types=[
            pltpu.VMEM((256,), jnp.int32),
            pltpu.VMEM((2, _L), jnp.int32),
            pltpu.VMEM((32, _L), jnp.int32),
            pltpu.VMEM_SHARED((32, _L), jnp.int32),
            pltpu.VMEM((4, 32), jnp.int32),
            pltpu.VMEM((4, 32), jnp.int32),
            pltpu.VMEM((2, 32, D), jnp.float32),
            pltpu.VMEM((NTP,), jnp.int32),
            pltpu.VMEM((_L,), jnp.float32),
            pltpu.VMEM((1, _L), jnp.float32),
            pltpu.SemaphoreType.DMA,
            pltpu.SemaphoreType.DMA,
        ],
        compiler_params=_sc_compiler_params(),
    )
    return kern(eidx_flat, xbf, psum)


def _ffn_body(te_ref, xs_ref, w1_ref, w3_ref, w2_ref, ys_ref):
    xb = xs_ref[...].astype(jnp.bfloat16)
    a = lax.dot_general(xb, w1_ref[0].astype(jnp.bfloat16),
                        (((1,), (1,)), ((), ())),
                        preferred_element_type=jnp.float32)
    b = lax.dot_general(xb, w3_ref[0].astype(jnp.bfloat16),
                        (((1,), (1,)), ((), ())),
                        preferred_element_type=jnp.float32)
    h = (a * jax.nn.sigmoid(a)) * b
    hb = h.astype(jnp.bfloat16)
    y = lax.dot_general(hb, w2_ref[0].astype(jnp.bfloat16),
                        (((1,), (1,)), ((), ())),
                        preferred_element_type=jnp.float32)
    ys_ref[...] = y


def _ffn(te, xs, w1b, w3b, w2b):
    grid_spec = pltpu.PrefetchScalarGridSpec(
        num_scalar_prefetch=1,
        grid=(NT,),
        in_specs=[
            pl.BlockSpec((TILE, D), lambda i, te: (i, 0)),
            pl.BlockSpec((1, F, D), lambda i, te: (te[i], 0, 0)),
            pl.BlockSpec((1, F, D), lambda i, te: (te[i], 0, 0)),
            pl.BlockSpec((1, D, F), lambda i, te: (te[i], 0, 0)),
        ],
        out_specs=pl.BlockSpec((TILE, D), lambda i, te: (i, 0)),
    )
    return pl.pallas_call(
        _ffn_body,
        grid_spec=grid_spec,
        out_shape=jax.ShapeDtypeStruct((P, D), jnp.float32),
        compiler_params=pltpu.CompilerParams(
            dimension_semantics=("arbitrary",)),
    )(te, xs, w1b, w3b, w2b)


def _combine_body(ys_hbm, pos_hbm, wts_hbm, out_hbm,
                  idx_vm, w_vm, abuf, bbuf):
    c = lax.axis_index("c")
    s = lax.axis_index("s")
    wid = 2 * s + c
    iota = _iota16()
    fz = jnp.zeros((_L,), jnp.float32)
    for g in range(2):
        tg = wid * 64 + g * 32
        pltpu.sync_copy(pos_hbm.at[pl.ds(tg, 32)], idx_vm.at[0])
        pltpu.sync_copy(pos_hbm.at[pl.ds(N + tg, 32)], idx_vm.at[1])
        pltpu.sync_copy(wts_hbm.at[pl.ds(tg, 32)], w_vm.at[0])
        pltpu.sync_copy(wts_hbm.at[pl.ds(N + tg, 32)], w_vm.at[1])
        pltpu.sync_copy(ys_hbm.at[idx_vm.at[0]], abuf)
        pltpu.sync_copy(ys_hbm.at[idx_vm.at[1]], bbuf)

        wv = [[w_vm[k, pl.ds(half * _L, _L)] for half in range(2)]
              for k in range(2)]
        for l in range(32):
            lane, half = l % _L, l // _L
            w0s = jnp.sum(jnp.where(iota == lane, wv[0][half], fz))
            w1s = jnp.sum(jnp.where(iota == lane, wv[1][half], fz))
            w0 = jnp.full((_L,), w0s, jnp.float32)
            w1 = jnp.full((_L,), w1s, jnp.float32)

            @pl.loop(0, D // (8 * _L))
            def _(cg, l=l, w0=w0, w1=w1):
                for c8 in range(8):
                    slc = pl.ds(cg * 8 * _L + c8 * _L, _L)
                    abuf[l, slc] = abuf[l, slc] * w0 + bbuf[l, slc] * w1

        pltpu.sync_copy(abuf, out_hbm.at[pl.ds(tg, 32)])


def _combine(ys, pos_flat, wts_flat):
    mesh = plsc.VectorSubcoreMesh(core_axis_name="c", subcore_axis_name="s")
    kern = pl.kernel(
        _combine_body,
        out_type=jax.ShapeDtypeStruct((N, D), jnp.float32),
        mesh=mesh,
        scratch_types=[
            pltpu.VMEM((2, 32), jnp.int32),
            pltpu.VMEM((2, 32), jnp.float32),
            pltpu.VMEM((32, D), jnp.float32),
            pltpu.VMEM((32, D), jnp.float32),
        ],
        compiler_params=_sc_compiler_params(),
    )
    return kern(ys, pos_flat, wts_flat)


def kernel(x, Wr, w1, w3, w2):
    B, T, Dm = x.shape
    xt = x.reshape(B * T, Dm)
    logits, eidx, wts, psum = _router(xt, Wr)
    pos, te, xs, aux = _dispatch(eidx.reshape(K * N), xt, psum)
    ys = _ffn(te, xs, w1, w3, w2)
    out = _combine(ys, pos.reshape(K * N), wts.reshape(K * N))
    return (out.reshape(B, T, Dm), aux[0], logits.reshape(B, T, E))


# speedup vs baseline: 1.7586x; 1.4139x over previous
import dataclasses
import functools

import jax
import jax.numpy as jnp
from jax import lax
from jax.experimental import pallas as pl
from jax.experimental.pallas import tpu as pltpu
from jax.experimental.pallas import tpu_sc as plsc

E = 8
K = 2
TILE = 256
ALPHA = 0.01
N = 2048
D = 1024
F = 2048
NT = (K * N) // TILE + E
P = NT * TILE
NTP = 32
_TSH = TILE.bit_length() - 1

def _sc_compiler_params():
    cp = pltpu.CompilerParams()
    if "needs_layout_passes" in pltpu.CompilerParams.__dataclass_fields__:
        cp = dataclasses.replace(cp, needs_layout_passes=False)
    return cp


_NC = 2
_NS = 16
_L = 16


def _iota16():
    return lax.iota(jnp.int32, _L)


def _full16(v, dtype=jnp.int32):
    return jnp.full((_L,), v, dtype=dtype)


def _router_body(x_ref, wr_ref, logits_ref, eidx_ref, wts_ref, psum_ref):
    x = x_ref[...]
    wr = wr_ref[...]
    logits = lax.dot_general(
        x.astype(jnp.bfloat16), wr.astype(jnp.bfloat16),
        (((1,), (1,)), ((), ())),
        preferred_element_type=jnp.float32)
    logits_ref[...] = logits
    io = lax.broadcasted_iota(jnp.int32, logits.shape, 1)
    v0 = jnp.max(logits, axis=1, keepdims=True)
    i0 = jnp.min(jnp.where(logits == v0, io, E), axis=1, keepdims=True)
    m0 = io == i0
    l1 = jnp.where(m0, -jnp.inf, logits)
    v1 = jnp.max(l1, axis=1, keepdims=True)
    i1 = jnp.min(jnp.where(l1 == v1, io, E), axis=1, keepdims=True)
    ex = jnp.exp(v1 - v0)
    w0 = 1.0 / (1.0 + ex)
    w1 = ex / (1.0 + ex)
    eidx_ref[...] = jnp.concatenate([i0, i1], axis=1).T.astype(jnp.int32)
    wts_ref[...] = jnp.concatenate([w0, w1], axis=1).T
    pm = jnp.exp(logits - v0)
    probs = pm / jnp.sum(pm, axis=1, keepdims=True)
    psum = jnp.sum(probs, axis=0, keepdims=True)
    psum_ref[...] = jnp.concatenate(
        [psum, jnp.zeros((1, _L - E), jnp.float32)], axis=1)


def _router(xt, wr):
    return pl.pallas_call(
        _router_body,
        out_shape=(
            jax.ShapeDtypeStruct((N, E), jnp.float32),
            jax.ShapeDtypeStruct((K, N), jnp.int32),
            jax.ShapeDtypeStruct((K, N), jnp.float32),
            jax.ShapeDtypeStruct((1, _L), jnp.float32),
        ),
    )(xt, wr)


def _dispatch_body(eidx_hbm, xbf_hbm, psum_hbm, pos_hbm, te_hbm, xs_hbm,
                  aux_hbm, e2_vm, hist_vm, hists_all, shared, posbuf, tokbuf,
                  rowbuf, tebuf, auxbuf, psum_vm, gsem, ssem):
    c = lax.axis_index("c")
    s = lax.axis_index("s")
    mychunk = 2 * s + c
    iota = _iota16()

    pltpu.sync_copy(eidx_hbm.at[pl.ds(s * 256, 256)], e2_vm)
    for h in range(2):
        hv = jnp.zeros((_L,), jnp.int32)
        for v in range(8):
            ev = e2_vm[pl.ds(h * 128 + v * _L, _L)]
            for e in range(E):
                cnt = jnp.sum(jnp.where(ev == e, 1, 0))
                hv = hv + jnp.where(iota == e, _full16(cnt), _full16(0))
        hist_vm[h, pl.ds(0, _L)] = hv
    pltpu.sync_copy(hist_vm, shared.at[pl.ds(2 * s, 2)])
    plsc.subcore_barrier()

    pltpu.sync_copy(shared, hists_all)
    counts_vec = jnp.zeros((_L,), jnp.int32)
    prior_vec = jnp.zeros((_L,), jnp.int32)
    for w in range(32):
        hv = hists_all[w, pl.ds(0, _L)]
        counts_vec = counts_vec + hv
        take = jnp.full((_L,), w < mychunk)
        prior_vec = prior_vec + jnp.where(take, hv, _full16(0))
    cnt_s = [jnp.sum(jnp.where(iota == e, counts_vec, _full16(0)))
             for e in range(E)]
    prior_s = [jnp.sum(jnp.where(iota == e, prior_vec, _full16(0)))
               for e in range(E)]
    po = []
    run = jnp.int32(0)
    for e in range(E):
        po.append(run)
        aligned = ((cnt_s[e] + (TILE - 1)) >> _TSH) << _TSH
        run = run + aligned
    tbl = [po[e] + prior_s[e] for e in range(E)]

    for v in range(8):
        ev = e2_vm[pl.ds(c * 128 + v * _L, _L)]
        pos_v = jnp.zeros((_L,), jnp.int32)
        for e in range(E):
            m = ev == e
            mi = jnp.where(m, 1, 0).astype(jnp.int32)
            cs = jnp.cumsum(mi)
            pos_v = pos_v + jnp.where(m, _full16(tbl[e]) + cs - 1, _full16(0))
            tbl[e] = tbl[e] + jnp.sum(mi)
        posbuf[v // 2, pl.ds((v % 2) * _L, _L)] = pos_v
        tok_v = (mychunk * 128 + v * _L + iota) & (N - 1)
        tokbuf[v // 2, pl.ds((v % 2) * _L, _L)] = tok_v
    pltpu.sync_copy(posbuf, pos_hbm.at[mychunk])

    pltpu.async_copy(xbf_hbm.at[tokbuf.at[0]], rowbuf.at[0], gsem).wait()
    for g in range(4):
        scp = pltpu.async_copy(rowbuf.at[g % 2], xs_hbm.at[posbuf.at[g]],
                               ssem)
        if g < 3:
            pltpu.async_copy(xbf_hbm.at[tokbuf.at[g + 1]],
                             rowbuf.at[(g + 1) % 2], gsem).wait()
        scp.wait()

    @pl.when(jnp.logical_and(c == 0, s == 0))
    def _():
        for grp in range(NTP // _L):
            tv = (grp * _L + iota) * TILE
            acc = jnp.zeros((_L,), jnp.int32)
            for e in range(E):
                acc = acc + jnp.where(tv >= _full16(po[e]),
                                      _full16(1), _full16(0))
            tebuf[pl.ds(grp * _L, _L)] = acc - 1
        pltpu.sync_copy(tebuf, te_hbm)
        pltpu.sync_copy(psum_hbm, psum_vm)
        pv = psum_vm[0, pl.ds(0, _L)]
        cf = counts_vec.astype(jnp.float32)
        total = jnp.sum(cf * pv)
        auxbuf[pl.ds(0, _L)] = _full16(
            total * (ALPHA * E / (N * N)), jnp.float32)
        pltpu.sync_copy(auxbuf, aux_hbm)


def _dispatch(eidx_flat, xbf, psum):
    mesh = plsc.VectorSubcoreMesh(core_axis_name="c", subcore_axis_name="s")
    kern = pl.kernel(
        _dispatch_body,
        out_type=(
            jax.ShapeDtypeStruct((32, 4, 32), jnp.int32),
            jax.ShapeDtypeStruct((NTP,), jnp.int32),
            jax.ShapeDtypeStruct((P, D), jnp.float32),
            jax.ShapeDtypeStruct((_L,), jnp.float32),
        ),
        mesh=mesh,
        scratch_types=[
            pltpu.VMEM((256,), jnp.int32),
            pltpu.VMEM((2, _L), jnp.int32),
            pltpu.VMEM((32, _L), jnp.int32),
            pltpu.VMEM_SHARED((32, _L), jnp.int32),
            pltpu.VMEM((4, 32), jnp.int32),
            pltpu.VMEM((4, 32), jnp.int32),
            pltpu.VMEM((2, 32, D), jnp.float32),
            pltpu.VMEM((NTP,), jnp.int32),
            pltpu.VMEM((_L,), jnp.float32),
            pltpu.VMEM((1, _L), jnp.float32),
            pltpu.SemaphoreType.DMA,
            pltpu.SemaphoreType.DMA,
        ],
        compiler_params=_sc_compiler_params(),
    )
    return kern(eidx_flat, xbf, psum)


def _ffn_body(te_ref, xs_ref, w1_ref, w3_ref, w2_ref, ys_ref):
    xb = xs_ref[...].astype(jnp.bfloat16)
    a = lax.dot_general(xb, w1_ref[0].astype(jnp.bfloat16),
                        (((1,), (1,)), ((), ())),
                        preferred_element_type=jnp.float32)
    b = lax.dot_general(xb, w3_ref[0].astype(jnp.bfloat16),
                        (((1,), (1,)), ((), ())),
                        preferred_element_type=jnp.float32)
    h = (a * jax.nn.sigmoid(a)) * b
    hb = h.astype(jnp.bfloat16)
    y = lax.dot_general(hb, w2_ref[0].astype(jnp.bfloat16),
                        (((1,), (1,)), ((), ())),
                        preferred_element_type=jnp.float32)
    ys_ref[...] = y


def _ffn(te, xs, w1b, w3b, w2b):
    grid_spec = pltpu.PrefetchScalarGridSpec(
        num_scalar_prefetch=1,
        grid=(NT,),
        in_specs=[
            pl.BlockSpec((TILE, D), lambda i, te: (i, 0)),
            pl.BlockSpec((1, F, D), lambda i, te: (te[i], 0, 0)),
            pl.BlockSpec((1, F, D), lambda i, te: (te[i], 0, 0)),
            pl.BlockSpec((1, D, F), lambda i, te: (te[i], 0, 0)),
        ],
        out_specs=pl.BlockSpec((TILE, D), lambda i, te: (i, 0)),
    )
    return pl.pallas_call(
        _ffn_body,
        grid_spec=grid_spec,
        out_shape=jax.ShapeDtypeStruct((P, D), jnp.float32),
        compiler_params=pltpu.CompilerParams(
            dimension_semantics=("arbitrary",)),
    )(te, xs, w1b, w3b, w2b)


def _combine_body(ys_hbm, pos_hbm, wts_hbm, out_hbm,
                  idx_vm, w_vm, abuf, bbuf):
    c = lax.axis_index("c")
    s = lax.axis_index("s")
    wid = 2 * s + c
    iota = _iota16()
    fz = jnp.zeros((_L,), jnp.float32)
    for g in range(2):
        tg = wid * 64 + g * 32
        pltpu.sync_copy(pos_hbm.at[pl.ds(tg, 32)], idx_vm.at[0])
        pltpu.sync_copy(pos_hbm.at[pl.ds(N + tg, 32)], idx_vm.at[1])
        pltpu.sync_copy(wts_hbm.at[pl.ds(tg, 32)], w_vm.at[0])
        pltpu.sync_copy(wts_hbm.at[pl.ds(N + tg, 32)], w_vm.at[1])
        pltpu.sync_copy(ys_hbm.at[idx_vm.at[0]], abuf)
        pltpu.sync_copy(ys_hbm.at[idx_vm.at[1]], bbuf)

        wv = [[w_vm[k, pl.ds(half * _L, _L)] for half in range(2)]
              for k in range(2)]
        for l in range(32):
            lane, half = l % _L, l // _L
            w0s = jnp.sum(jnp.where(iota == lane, wv[0][half], fz))
            w1s = jnp.sum(jnp.where(iota == lane, wv[1][half], fz))
            w0 = jnp.full((_L,), w0s, jnp.float32)
            w1 = jnp.full((_L,), w1s, jnp.float32)

            @pl.loop(0, D // (8 * _L))
            def _(cg, l=l, w0=w0, w1=w1):
                for c8 in range(8):
                    slc = pl.ds(cg * 8 * _L + c8 * _L, _L)
                    abuf[l, slc] = abuf[l, slc] * w0 + bbuf[l, slc] * w1

        pltpu.sync_copy(abuf, out_hbm.at[pl.ds(tg, 32)])


def _combine(ys, pos_flat, wts_flat):
    mesh = plsc.VectorSubcoreMesh(core_axis_name="c", subcore_axis_name="s")
    kern = pl.kernel(
        _combine_body,
        out_type=jax.ShapeDtypeStruct((N, D), jnp.float32),
        mesh=mesh,
        scratch_types=[
            pltpu.VMEM((2, 32), jnp.int32),
            pltpu.VMEM((2, 32), jnp.float32),
            pltpu.VMEM((32, D), jnp.float32),
            pltpu.VMEM((32, D), jnp.float32),
        ],
        compiler_params=_sc_compiler_params(),
    )
    return kern(ys, pos_flat, wts_flat)


def kernel(x, Wr, w1, w3, w2):
    B, T, Dm = x.shape
    xt = x.reshape(B * T, Dm)
    logits, eidx, wts, psum = _router(xt, Wr)
    pos, te, xs, aux = _dispatch(eidx.reshape(K * N), xt, psum)
    ys = _ffn(te, xs, w1, w3, w2)
    out = _combine(ys, pos.reshape(K * N), wts.reshape(K * N))
    return (out.reshape(B, T, Dm), aux[0], logits.reshape(B, T, E))
